# baseline (speedup 1.0000x reference)
import functools

import jax
import jax.numpy as jnp
from jax import lax
from jax.experimental import pallas as pl
from jax.experimental.pallas import tpu as pltpu
from jax.experimental.pallas import tpu_sc as plsc

VOCAB = 100000
H = 768
HW = H // 2
POS = 4096
B = 4
S = 4096

L = 16
NP = H // 32
NW = 32
ROWS = B * S
RPW = ROWS // NW
CH = 32
NBUF = 2
NCH = RPW // CH
NPAIR = NCH // NBUF
SCALE = float(H) ** 0.5
EPS = 1e-5

TR = 512
NTB = ROWS // TR
SB = S // TR

_mesh = plsc.VectorSubcoreMesh(core_axis_name="c", subcore_axis_name="s")

_GDN = lax.GatherDimensionNumbers(
    offset_dims=(), collapsed_slice_dims=(0,), start_index_map=(0,))


def _perm(v, idx):
    return lax.gather(v, idx.reshape(L, 1), _GDN, slice_sizes=(1,),
                      mode=lax.GatherScatterMode.PROMISE_IN_BOUNDS)


@functools.partial(
    pl.kernel,
    out_type=jax.ShapeDtypeStruct((ROWS, HW), jnp.int32),
    mesh=_mesh,
    compiler_params=pltpu.CompilerParams(needs_layout_passes=False),
    scratch_types=[
        pltpu.VMEM((RPW,), jnp.int32),
        pltpu.VMEM((NBUF, CH, H), jnp.float32),
        pltpu.VMEM((NBUF, CH, HW), jnp.int32),
        [pltpu.SemaphoreType.DMA] * NBUF,
        [pltpu.SemaphoreType.DMA] * NBUF,
    ],
)
def _gather_kernel(ids_hbm, ww_hbm, out_hbm, idx_v, rows_v, pk_v, gsems, osems):
    wid = lax.axis_index("s") * 2 + lax.axis_index("c")
    base = wid * RPW
    pltpu.sync_copy(ids_hbm.at[pl.ds(base, RPW)], idx_v)
    for p in range(NBUF):
        pltpu.async_copy(
            ww_hbm.at[idx_v.at[pl.ds(p * CH, CH)]], rows_v.at[p], gsems[p])

    lanes = lax.iota(jnp.int32, L)
    idx_e = (lanes * 2) & (L - 1)
    idx_o = (lanes * 2 + 1) & (L - 1)
    lo_half = lanes < (L // 2)

    def _pack_rows(b):
        def body(i, carry):
            for m in range(NP):
                v0 = rows_v[b, i, pl.ds(32 * m, L)]
                v1 = rows_v[b, i, pl.ds(32 * m + L, L)]
                ev = jnp.where(lo_half, _perm(v0, idx_e), _perm(v1, idx_e))
                od = jnp.where(lo_half, _perm(v0, idx_o), _perm(v1, idx_o))
                ie = plsc.bitcast(ev, jnp.int32) + 0x8000
                io = plsc.bitcast(od, jnp.int32) + 0x8000
                word = (lax.shift_right_logical(ie, 16)
                        | (io & jnp.int32(-65536)))
                pk_v[b, i, pl.ds(L * m, L)] = word
            return carry
        lax.fori_loop(0, CH, body, 0)

    def pair_body(k, carry):
        for p in range(NBUF):
            c = k * NBUF + p
            pltpu.make_async_copy(
                ww_hbm.at[idx_v.at[pl.ds(0, CH)]], rows_v.at[p],
                gsems[p]).wait()

            @pl.when(k > 0)
            def _():
                pltpu.make_async_copy(
                    pk_v.at[p], out_hbm.at[pl.ds(base, CH)], osems[p]).wait()

            _pack_rows(p)
            pltpu.async_copy(
                pk_v.at[p], out_hbm.at[pl.ds(base + c * CH, CH)], osems[p])

            @pl.when(c + NBUF < NCH)
            def _():
                pltpu.async_copy(
                    ww_hbm.at[idx_v.at[pl.ds((c + NBUF) * CH, CH)]],
                    rows_v.at[p], gsems[p])
        return carry

    lax.fori_loop(0, NPAIR, pair_body, 0)
    for p in range(NBUF):
        pltpu.make_async_copy(
            pk_v.at[p], out_hbm.at[pl.ds(base, CH)], osems[p]).wait()


def _ln_body(g_ref, p_ref, s_ref, ws_ref, ga_ref, be_ref, o_ref):
    x = g_ref[...].astype(jnp.float32) * SCALE + p_ref[...]
    sidf = s_ref[0, 0, :].astype(jnp.float32)[:, None]
    x = x + ws_ref[0:1, :] + sidf * (ws_ref[1:2, :] - ws_ref[0:1, :])
    mu = jnp.mean(x, axis=-1, keepdims=True)
    var = jnp.mean(x * x, axis=-1, keepdims=True) - mu * mu
    o_ref[...] = (x - mu) * lax.rsqrt(var + EPS) * ga_ref[...] + be_ref[...]


_ln_call = pl.pallas_call(
    _ln_body,
    grid=(NTB,),
    in_specs=[
        pl.BlockSpec((TR, H), lambda i: ((i % B) * SB + i // B, 0)),
        pl.BlockSpec((TR, H), lambda i: (i // B, 0)),
        pl.BlockSpec((1, 1, TR), lambda i: ((i % B) * SB + i // B, 0, 0)),
        pl.BlockSpec((2, H), lambda i: (0, 0)),
        pl.BlockSpec((1, H), lambda i: (0, 0)),
        pl.BlockSpec((1, H), lambda i: (0, 0)),
    ],
    out_specs=pl.BlockSpec((TR, H), lambda i: ((i % B) * SB + i // B, 0)),
    out_shape=jax.ShapeDtypeStruct((ROWS, H), jnp.float32),
)


def kernel(input_ids, segment_ids, W_word, W_seg, gamma, beta, pos_enc):
    ids = input_ids.reshape(ROWS).astype(jnp.int32)
    seg3 = segment_ids.reshape(NTB, 1, TR).astype(jnp.int32)
    packed = _gather_kernel(ids, W_word)
    st_bf16 = lax.bitcast_convert_type(packed, jnp.bfloat16).reshape(ROWS, H)
    out = _ln_call(st_bf16, pos_enc, seg3, W_seg,
                   gamma.reshape(1, H), beta.reshape(1, H))
    return out.reshape(B, S, H)


# speedup vs baseline: 2.4372x; 2.4372x over previous
import functools

import jax
import jax.numpy as jnp
from jax import lax
from jax.experimental import pallas as pl
from jax.experimental.pallas import tpu as pltpu
from jax.experimental.pallas import tpu_sc as plsc

VOCAB = 100000
H = 768
HW = H // 2
POS = 4096
B = 4
S = 4096

L = 16
NG = HW // L
NW = 32
ROWS = B * S
RPW = ROWS // NW
CH = 32
NBUF = 2
NCH = RPW // CH
NPAIR = NCH // NBUF
SCALE = float(H) ** 0.5
EPS = 1e-5

TR = 512
NTB = ROWS // TR
SB = S // TR

_mesh = plsc.VectorSubcoreMesh(core_axis_name="c", subcore_axis_name="s")


@functools.partial(
    pl.kernel,
    out_type=jax.ShapeDtypeStruct((ROWS, HW), jnp.int32),
    mesh=_mesh,
    compiler_params=pltpu.CompilerParams(needs_layout_passes=False),
    scratch_types=[
        pltpu.VMEM((RPW,), jnp.int32),
        pltpu.VMEM((NBUF, CH, H), jnp.float32),
        pltpu.VMEM((NBUF, CH, HW), jnp.int32),
        [pltpu.SemaphoreType.DMA] * NBUF,
        [pltpu.SemaphoreType.DMA] * NBUF,
    ],
)
def _gather_kernel(ids_hbm, ww_hbm, out_hbm, idx_v, rows_v, pk_v, gsems, osems):
    wid = lax.axis_index("s") * 2 + lax.axis_index("c")
    base = wid * RPW
    pltpu.sync_copy(ids_hbm.at[pl.ds(base, RPW)], idx_v)
    for p in range(NBUF):
        pltpu.async_copy(
            ww_hbm.at[idx_v.at[pl.ds(p * CH, CH)]], rows_v.at[p], gsems[p])

    def _pack_rows(b):
        def body(i, carry):
            for m in range(NG):
                v0 = rows_v[b, i, pl.ds(L * m, L)]
                v1 = rows_v[b, i, pl.ds(HW + L * m, L)]
                i0 = plsc.bitcast(v0, jnp.int32) + 0x8000
                i1 = plsc.bitcast(v1, jnp.int32) + 0x8000
                word = (lax.shift_right_logical(i0, 16)
                        | (i1 & jnp.int32(-65536)))
                pk_v[b, i, pl.ds(L * m, L)] = word
            return carry
        lax.fori_loop(0, CH, body, 0)

    def pair_body(k, carry):
        for p in range(NBUF):
            c = k * NBUF + p
            pltpu.make_async_copy(
                ww_hbm.at[idx_v.at[pl.ds(0, CH)]], rows_v.at[p],
                gsems[p]).wait()

            @pl.when(k > 0)
            def _():
                pltpu.make_async_copy(
                    pk_v.at[p], out_hbm.at[pl.ds(base, CH)], osems[p]).wait()

            _pack_rows(p)
            pltpu.async_copy(
                pk_v.at[p], out_hbm.at[pl.ds(base + c * CH, CH)], osems[p])

            @pl.when(c + NBUF < NCH)
            def _():
                pltpu.async_copy(
                    ww_hbm.at[idx_v.at[pl.ds((c + NBUF) * CH, CH)]],
                    rows_v.at[p], gsems[p])
        return carry

    lax.fori_loop(0, NPAIR, pair_body, 0)
    for p in range(NBUF):
        pltpu.make_async_copy(
            pk_v.at[p], out_hbm.at[pl.ds(base, CH)], osems[p]).wait()


def _ln_body(g_ref, p_ref, s_ref, ws_ref, ga_ref, be_ref, o_ref):
    w = g_ref[...]
    xa = lax.bitcast_convert_type(w << 16, jnp.float32)
    xb = lax.bitcast_convert_type(w & jnp.int32(-65536), jnp.float32)
    x = jnp.concatenate([xa, xb], axis=-1) * SCALE + p_ref[...]
    sidf = s_ref[0, 0, :].astype(jnp.float32)[:, None]
    x = x + ws_ref[0:1, :] + sidf * (ws_ref[1:2, :] - ws_ref[0:1, :])
    mu = jnp.mean(x, axis=-1, keepdims=True)
    var = jnp.mean(x * x, axis=-1, keepdims=True) - mu * mu
    o_ref[...] = (x - mu) * lax.rsqrt(var + EPS) * ga_ref[...] + be_ref[...]


_ln_call = pl.pallas_call(
    _ln_body,
    grid=(NTB,),
    in_specs=[
        pl.BlockSpec((TR, HW), lambda i: ((i % B) * SB + i // B, 0)),
        pl.BlockSpec((TR, H), lambda i: (i // B, 0)),
        pl.BlockSpec((1, 1, TR), lambda i: ((i % B) * SB + i // B, 0, 0)),
        pl.BlockSpec((2, H), lambda i: (0, 0)),
        pl.BlockSpec((1, H), lambda i: (0, 0)),
        pl.BlockSpec((1, H), lambda i: (0, 0)),
    ],
    out_specs=pl.BlockSpec((TR, H), lambda i: ((i % B) * SB + i // B, 0)),
    out_shape=jax.ShapeDtypeStruct((ROWS, H), jnp.float32),
)


def kernel(input_ids, segment_ids, W_word, W_seg, gamma, beta, pos_enc):
    ids = input_ids.reshape(ROWS).astype(jnp.int32)
    seg3 = segment_ids.reshape(NTB, 1, TR).astype(jnp.int32)
    packed = _gather_kernel(ids, W_word)
    out = _ln_call(packed, pos_enc, seg3, W_seg,
                   gamma.reshape(1, H), beta.reshape(1, H))
    return out.reshape(B, S, H)


# speedup vs baseline: 2.7041x; 1.1095x over previous
import functools

import jax
import jax.numpy as jnp
from jax import lax
from jax.experimental import pallas as pl
from jax.experimental.pallas import tpu as pltpu
from jax.experimental.pallas import tpu_sc as plsc

VOCAB = 100000
H = 768
POS = 4096
B = 4
S = 4096

NW = 32
ROWS = B * S
RPW = ROWS // NW
CH = 32
NBUF = 4
NCH = RPW // CH
SCALE = float(H) ** 0.5
EPS = 1e-5

TR = 256
NTB = ROWS // TR
SB = S // TR

_mesh = plsc.VectorSubcoreMesh(core_axis_name="c", subcore_axis_name="s")


@functools.partial(
    pl.kernel,
    out_type=jax.ShapeDtypeStruct((ROWS, H), jnp.float32),
    mesh=_mesh,
    compiler_params=pltpu.CompilerParams(needs_layout_passes=False),
    scratch_types=[
        pltpu.VMEM((RPW,), jnp.int32),
        pltpu.VMEM((NBUF, CH, H), jnp.float32),
        [pltpu.SemaphoreType.DMA] * NBUF,
    ],
)
def _gather_kernel(ids_hbm, ww_hbm, out_hbm, idx_v, rows_v, sems):
    wid = lax.axis_index("s") * 2 + lax.axis_index("c")
    base = wid * RPW
    pltpu.sync_copy(ids_hbm.at[pl.ds(base, RPW)], idx_v)
    copies = [None] * NBUF
    for p in range(NBUF):
        copies[p] = pltpu.async_copy(
            ww_hbm.at[idx_v.at[pl.ds(p * CH, CH)]], rows_v.at[p], sems[p])
    for c in range(NCH):
        b = c % NBUF
        copies[b].wait()
        pltpu.sync_copy(rows_v.at[b], out_hbm.at[pl.ds(base + c * CH, CH)])
        if c + NBUF < NCH:
            copies[b] = pltpu.async_copy(
                ww_hbm.at[idx_v.at[pl.ds((c + NBUF) * CH, CH)]],
                rows_v.at[b], sems[b])


def _ln_body(g_ref, p_ref, s_ref, ws_ref, ga_ref, be_ref, o_ref):
    x = g_ref[...] * SCALE + p_ref[...]
    sidf = s_ref[0, 0, :].astype(jnp.float32)[:, None]
    x = x + ws_ref[0:1, :] + sidf * (ws_ref[1:2, :] - ws_ref[0:1, :])
    mu = jnp.mean(x, axis=-1, keepdims=True)
    var = jnp.mean(x * x, axis=-1, keepdims=True) - mu * mu
    o_ref[...] = (x - mu) * lax.rsqrt(var + EPS) * ga_ref[...] + be_ref[...]


_ln_call = pl.pallas_call(
    _ln_body,
    grid=(NTB,),
    in_specs=[
        pl.BlockSpec((TR, H), lambda i: ((i % B) * SB + i // B, 0)),
        pl.BlockSpec((TR, H), lambda i: (i // B, 0)),
        pl.BlockSpec((1, 1, TR), lambda i: ((i % B) * SB + i // B, 0, 0)),
        pl.BlockSpec((2, H), lambda i: (0, 0)),
        pl.BlockSpec((1, H), lambda i: (0, 0)),
        pl.BlockSpec((1, H), lambda i: (0, 0)),
    ],
    out_specs=pl.BlockSpec((TR, H), lambda i: ((i % B) * SB + i // B, 0)),
    out_shape=jax.ShapeDtypeStruct((ROWS, H), jnp.float32),
)


def kernel(input_ids, segment_ids, W_word, W_seg, gamma, beta, pos_enc):
    ids = input_ids.reshape(ROWS).astype(jnp.int32)
    seg3 = segment_ids.reshape(NTB, 1, TR).astype(jnp.int32)
    gathered = _gather_kernel(ids, W_word)
    out = _ln_call(gathered, pos_enc, seg3, W_seg,
                   gamma.reshape(1, H), beta.reshape(1, H))
    return out.reshape(B, S, H)


# speedup vs baseline: 3.4804x; 1.2871x over previous
import functools

import jax
import jax.numpy as jnp
from jax import lax
from jax.experimental import pallas as pl
from jax.experimental.pallas import tpu as pltpu
from jax.experimental.pallas import tpu_sc as plsc

VOCAB = 100000
H = 768
POS = 4096
B = 4
S = 4096

NW = 32
ROWS = B * S
RPW = ROWS // NW
CH = 32
NBUF = 4
NCH = RPW // CH
SCALE = float(H) ** 0.5
EPS = 1e-5

TR = 1024
NTB = ROWS // TR
SB = S // TR

_mesh = plsc.VectorSubcoreMesh(core_axis_name="c", subcore_axis_name="s")


@functools.partial(
    pl.kernel,
    out_type=jax.ShapeDtypeStruct((ROWS, H), jnp.float32),
    mesh=_mesh,
    compiler_params=pltpu.CompilerParams(needs_layout_passes=False),
    scratch_types=[
        pltpu.VMEM((RPW,), jnp.int32),
        pltpu.VMEM((NBUF, CH, H), jnp.float32),
        [pltpu.SemaphoreType.DMA] * NBUF,
    ],
)
def _gather_kernel(ids_hbm, ww_hbm, out_hbm, idx_v, rows_v, sems):
    wid = lax.axis_index("s") * 2 + lax.axis_index("c")
    base = wid * RPW
    pltpu.sync_copy(ids_hbm.at[pl.ds(base, RPW)], idx_v)
    copies = [None] * NBUF
    for p in range(NBUF):
        copies[p] = pltpu.async_copy(
            ww_hbm.at[idx_v.at[pl.ds(p * CH, CH)]], rows_v.at[p], sems[p])
    for c in range(NCH):
        b = c % NBUF
        copies[b].wait()
        pltpu.sync_copy(rows_v.at[b], out_hbm.at[pl.ds(base + c * CH, CH)])
        if c + NBUF < NCH:
            copies[b] = pltpu.async_copy(
                ww_hbm.at[idx_v.at[pl.ds((c + NBUF) * CH, CH)]],
                rows_v.at[b], sems[b])


def _ln_body(g_ref, p_ref, s_ref, ws_ref, ga_ref, be_ref, o_ref):
    x = g_ref[...] * SCALE + p_ref[...]
    sidf = s_ref[0, 0, :].astype(jnp.float32)[:, None]
    x = x + ws_ref[0:1, :] + sidf * (ws_ref[1:2, :] - ws_ref[0:1, :])
    mu = jnp.mean(x, axis=-1, keepdims=True)
    var = jnp.mean(x * x, axis=-1, keepdims=True) - mu * mu
    o_ref[...] = (x - mu) * lax.rsqrt(var + EPS) * ga_ref[...] + be_ref[...]


_ln_call = pl.pallas_call(
    _ln_body,
    grid=(NTB,),
    in_specs=[
        pl.BlockSpec((TR, H), lambda i: ((i % B) * SB + i // B, 0)),
        pl.BlockSpec((TR, H), lambda i: (i // B, 0)),
        pl.BlockSpec((1, 1, TR), lambda i: ((i % B) * SB + i // B, 0, 0)),
        pl.BlockSpec((2, H), lambda i: (0, 0)),
        pl.BlockSpec((1, H), lambda i: (0, 0)),
        pl.BlockSpec((1, H), lambda i: (0, 0)),
    ],
    out_specs=pl.BlockSpec((TR, H), lambda i: ((i % B) * SB + i // B, 0)),
    out_shape=jax.ShapeDtypeStruct((ROWS, H), jnp.float32),
)


def kernel(input_ids, segment_ids, W_word, W_seg, gamma, beta, pos_enc):
    ids = input_ids.reshape(ROWS).astype(jnp.int32)
    seg3 = segment_ids.reshape(NTB, 1, TR).astype(jnp.int32)
    gathered = _gather_kernel(ids, W_word)
    out = _ln_call(gathered, pos_enc, seg3, W_seg,
                   gamma.reshape(1, H), beta.reshape(1, H))
    return out.reshape(B, S, H)


# speedup vs baseline: 3.5998x; 1.0343x over previous
import functools

import jax
import jax.numpy as jnp
from jax import lax
from jax.experimental import pallas as pl
from jax.experimental.pallas import tpu as pltpu
from jax.experimental.pallas import tpu_sc as plsc

VOCAB = 100000
H = 768
POS = 4096
B = 4
S = 4096

NW = 32
ROWS = B * S
RPW = ROWS // NW
CH = 32
NBUF = 4
NCH = RPW // CH
SCALE = float(H) ** 0.5
EPS = 1e-5

TR = 2048
NTB = ROWS // TR
SB = S // TR

_mesh = plsc.VectorSubcoreMesh(core_axis_name="c", subcore_axis_name="s")


@functools.partial(
    pl.kernel,
    out_type=jax.ShapeDtypeStruct((ROWS, H), jnp.float32),
    mesh=_mesh,
    compiler_params=pltpu.CompilerParams(needs_layout_passes=False),
    scratch_types=[
        pltpu.VMEM((RPW,), jnp.int32),
        pltpu.VMEM((NBUF, CH, H), jnp.float32),
        [pltpu.SemaphoreType.DMA] * NBUF,
    ],
)
def _gather_kernel(ids_hbm, ww_hbm, out_hbm, idx_v, rows_v, sems):
    wid = lax.axis_index("s") * 2 + lax.axis_index("c")
    base = wid * RPW
    pltpu.sync_copy(ids_hbm.at[pl.ds(base, RPW)], idx_v)
    copies = [None] * NBUF
    for p in range(NBUF):
        copies[p] = pltpu.async_copy(
            ww_hbm.at[idx_v.at[pl.ds(p * CH, CH)]], rows_v.at[p], sems[p])
    for c in range(NCH):
        b = c % NBUF
        copies[b].wait()
        pltpu.sync_copy(rows_v.at[b], out_hbm.at[pl.ds(base + c * CH, CH)])
        if c + NBUF < NCH:
            copies[b] = pltpu.async_copy(
                ww_hbm.at[idx_v.at[pl.ds((c + NBUF) * CH, CH)]],
                rows_v.at[b], sems[b])


def _ln_body(g_ref, p_ref, s_ref, ws_ref, ga_ref, be_ref, o_ref):
    x = g_ref[...] * SCALE + p_ref[...]
    sidf = s_ref[0, 0, :].astype(jnp.float32)[:, None]
    x = x + ws_ref[0:1, :] + sidf * (ws_ref[1:2, :] - ws_ref[0:1, :])
    mu = jnp.mean(x, axis=-1, keepdims=True)
    var = jnp.mean(x * x, axis=-1, keepdims=True) - mu * mu
    o_ref[...] = (x - mu) * lax.rsqrt(var + EPS) * ga_ref[...] + be_ref[...]


_ln_call = pl.pallas_call(
    _ln_body,
    grid=(NTB,),
    in_specs=[
        pl.BlockSpec((TR, H), lambda i: ((i % B) * SB + i // B, 0)),
        pl.BlockSpec((TR, H), lambda i: (i // B, 0)),
        pl.BlockSpec((1, 1, TR), lambda i: ((i % B) * SB + i // B, 0, 0)),
        pl.BlockSpec((2, H), lambda i: (0, 0)),
        pl.BlockSpec((1, H), lambda i: (0, 0)),
        pl.BlockSpec((1, H), lambda i: (0, 0)),
    ],
    out_specs=pl.BlockSpec((TR, H), lambda i: ((i % B) * SB + i // B, 0)),
    out_shape=jax.ShapeDtypeStruct((ROWS, H), jnp.float32),
)


def kernel(input_ids, segment_ids, W_word, W_seg, gamma, beta, pos_enc):
    ids = input_ids.reshape(ROWS).astype(jnp.int32)
    seg3 = segment_ids.reshape(NTB, 1, TR).astype(jnp.int32)
    gathered = _gather_kernel(ids, W_word)
    out = _ln_call(gathered, pos_enc, seg3, W_seg,
                   gamma.reshape(1, H), beta.reshape(1, H))
    return out.reshape(B, S, H)
